# baseline (speedup 1.0000x reference)
import jax
import jax.numpy as jnp
from jax import lax
from jax.experimental import pallas as pl
from jax.experimental.pallas import tpu as pltpu

_EPS = 1e-5


def _conv3(h, w_ref):
    L = h.shape[1]
    pos = lax.broadcasted_iota(jnp.int32, (1, L), 1)
    zero = jnp.zeros((), h.dtype)
    h_m1 = jnp.where(pos != 0, pltpu.roll(h, 1, axis=1), zero)
    h_p1 = jnp.where(pos != (L - 1), pltpu.roll(h, L - 1, axis=1), zero)
    out = jnp.dot(w_ref[0], h_m1, preferred_element_type=jnp.float32)
    out += jnp.dot(w_ref[1], h, preferred_element_type=jnp.float32)
    out += jnp.dot(w_ref[2], h_p1, preferred_element_type=jnp.float32)
    return out


def _accum_stats(y, n, s_ref, ss_ref):
    sl = s_ref.shape[1]
    ssq = y * y
    s = y[:, :sl]
    ss = ssq[:, :sl]
    for k in range(1, y.shape[1] // sl):
        s = s + y[:, k * sl:(k + 1) * sl]
        ss = ss + ssq[:, k * sl:(k + 1) * sl]
    s32 = s.astype(jnp.float32)
    ss32 = ss.astype(jnp.float32)

    @pl.when(n == 0)
    def _():
        s_ref[...] = s32
        ss_ref[...] = ss32

    @pl.when(n != 0)
    def _():
        s_ref[...] += s32
        ss_ref[...] += ss32


def _fold(s_ref, ss_ref, g_ref, b_ref, sc_ref, sh_ref, count, eps):
    mean = jnp.sum(s_ref[...], axis=1, keepdims=True) / count
    ex2 = jnp.sum(ss_ref[...], axis=1, keepdims=True) / count
    var = ex2 - mean * mean
    scale = g_ref[...] * lax.rsqrt(var + eps)
    sc_ref[...] = scale
    sh_ref[...] = b_ref[...] - mean * scale


def _make_fused_kernel(N, L, B, eps):
    cnt = float(N * L)

    def body(x_ref, g1_ref, b1_ref, g2_ref, b2_ref, w1_ref, w2_ref,
             o_ref,
             y_s, s_s, ss_s, s2_s, ss2_s,
             sc1_s, sh1_s, sc2_s, sh2_s):
        p = pl.program_id(0)
        t = pl.program_id(1)

        @pl.when(p == 0)
        def _phase1():
            for j in range(B):
                n = t * B + j
                xb = x_ref[j].astype(jnp.bfloat16)
                y1 = _conv3(xb, w1_ref).astype(jnp.bfloat16)
                y_s[n] = y1
                _accum_stats(y1, n, s_s, ss_s)

        @pl.when(jnp.logical_and(p == 1, t == 0))
        def _fold1():
            _fold(s_s, ss_s, g1_ref, b1_ref, sc1_s, sh1_s, cnt, eps)

        @pl.when(p == 1)
        def _phase2():
            sc = sc1_s[...].astype(jnp.bfloat16)
            sh = sh1_s[...].astype(jnp.bfloat16)
            zero = jnp.zeros((), jnp.bfloat16)
            for j in range(B):
                n = t * B + j
                h = jnp.maximum(y_s[n] * sc + sh, zero)
                y2 = _conv3(h, w2_ref).astype(jnp.bfloat16)
                y_s[n] = y2
                _accum_stats(y2, n, s2_s, ss2_s)

        @pl.when(jnp.logical_and(p == 2, t == 0))
        def _fold2():
            _fold(s2_s, ss2_s, g2_ref, b2_ref, sc2_s, sh2_s, cnt, eps)

        @pl.when(p == 2)
        def _phase3():
            for j in range(B):
                n = t * B + j
                o_ref[j] = jnp.maximum(
                    y_s[n].astype(jnp.float32) * sc2_s[...] + sh2_s[...]
                    + x_ref[j], 0.0)

    return body


def kernel(x_ncl, w1, g1, b1, w2, g2, b2):
    N, C, L = x_ncl.shape
    P = w1.shape[1]

    x_f = x_ncl.astype(jnp.float32)
    w1b = w1.astype(jnp.bfloat16)
    w2b = w2.astype(jnp.bfloat16)
    g1c = g1.astype(jnp.float32).reshape(P, 1)
    b1c = b1.astype(jnp.float32).reshape(P, 1)
    g2c = g2.astype(jnp.float32).reshape(P, 1)
    b2c = b2.astype(jnp.float32).reshape(P, 1)

    B = 8

    cparams = pltpu.CompilerParams(
        dimension_semantics=("arbitrary", "arbitrary"),
        vmem_limit_bytes=60 * 1024 * 1024)

    x_spec = pl.BlockSpec(
        (B, C, L), lambda p, t: (jnp.where(p == 1, 0, t), 0, 0))
    o_spec = pl.BlockSpec(
        (B, P, L), lambda p, t: (jnp.where(p == 2, t, 0), 0, 0))
    vec_spec = pl.BlockSpec((P, 1), lambda p, t: (0, 0))

    def w_spec(c_in):
        return pl.BlockSpec((3, P, c_in), lambda p, t: (0, 0, 0))

    stat_acc = pltpu.VMEM((P, 128), jnp.float32)
    stat = pltpu.VMEM((P, 1), jnp.float32)

    out = pl.pallas_call(
        _make_fused_kernel(N, L, B, _EPS),
        grid=(3, N // B),
        in_specs=[x_spec, vec_spec, vec_spec, vec_spec, vec_spec,
                  w_spec(C), w_spec(P)],
        out_specs=o_spec,
        out_shape=jax.ShapeDtypeStruct((N, P, L), jnp.float32),
        scratch_shapes=[
            pltpu.VMEM((N, P, L), jnp.bfloat16),
            stat_acc, stat_acc, stat_acc, stat_acc,
            stat, stat, stat, stat,
        ],
        compiler_params=cparams,
    )(x_f, g1c, b1c, g2c, b2c, w1b, w2b)

    return out


# speedup vs baseline: 1.0648x; 1.0648x over previous
import jax
import jax.numpy as jnp
from jax import lax
from jax.experimental import pallas as pl
from jax.experimental.pallas import tpu as pltpu

_EPS = 1e-5


def _edge_masks(L):
    pos = lax.broadcasted_iota(jnp.int32, (1, L), 1)
    mask_l = (pos != 0).astype(jnp.float32).astype(jnp.bfloat16)
    mask_r = (pos != (L - 1)).astype(jnp.float32).astype(jnp.bfloat16)
    return mask_l, mask_r


def _conv3(h, w_ref, mask_l, mask_r):
    L = h.shape[1]
    h_m1 = pltpu.roll(h, 1, axis=1) * mask_l
    h_p1 = pltpu.roll(h, L - 1, axis=1) * mask_r
    out = jnp.dot(w_ref[0], h_m1, preferred_element_type=jnp.float32)
    out += jnp.dot(w_ref[1], h, preferred_element_type=jnp.float32)
    out += jnp.dot(w_ref[2], h_p1, preferred_element_type=jnp.float32)
    return out


def _accum_stats(y, n, s_ref, ss_ref):
    sl = s_ref.shape[1]
    ssq = y * y
    s = y[:, :sl]
    ss = ssq[:, :sl]
    for k in range(1, y.shape[1] // sl):
        s = s + y[:, k * sl:(k + 1) * sl]
        ss = ss + ssq[:, k * sl:(k + 1) * sl]
    s32 = s.astype(jnp.float32)
    ss32 = ss.astype(jnp.float32)

    @pl.when(n == 0)
    def _():
        s_ref[...] = s32
        ss_ref[...] = ss32

    @pl.when(n != 0)
    def _():
        s_ref[...] += s32
        ss_ref[...] += ss32


def _fold(s_ref, ss_ref, g_ref, b_ref, sc_ref, sh_ref, count, eps):
    mean = jnp.sum(s_ref[...], axis=1, keepdims=True) / count
    ex2 = jnp.sum(ss_ref[...], axis=1, keepdims=True) / count
    var = ex2 - mean * mean
    scale = g_ref[...] * lax.rsqrt(var + eps)
    sc_ref[...] = scale
    sh_ref[...] = b_ref[...] - mean * scale


def _make_fused_kernel(N, L, B, eps):
    cnt = float(N * L)

    def body(x_ref, g1_ref, b1_ref, g2_ref, b2_ref, w1_ref, w2_ref,
             o_ref,
             xb_s, y_s, s_s, ss_s, s2_s, ss2_s,
             sc1_s, sh1_s, sc2_s, sh2_s):
        p = pl.program_id(0)
        t = pl.program_id(1)

        @pl.when(p == 0)
        def _phase1():
            mask_l, mask_r = _edge_masks(L)
            for j in range(B):
                n = t * B + j
                xb = x_ref[j].astype(jnp.bfloat16)
                xb_s[n] = xb
                y1 = _conv3(xb, w1_ref, mask_l, mask_r).astype(jnp.bfloat16)
                y_s[n] = y1
                _accum_stats(y1, n, s_s, ss_s)

        @pl.when(jnp.logical_and(p == 1, t == 0))
        def _fold1():
            _fold(s_s, ss_s, g1_ref, b1_ref, sc1_s, sh1_s, cnt, eps)

        @pl.when(p == 1)
        def _phase2():
            mask_l, mask_r = _edge_masks(L)
            sc = sc1_s[...].astype(jnp.bfloat16)
            sh = sh1_s[...].astype(jnp.bfloat16)
            zero = jnp.zeros((), jnp.bfloat16)
            for j in range(B):
                n = t * B + j
                h = jnp.maximum(y_s[n] * sc + sh, zero)
                y2 = _conv3(h, w2_ref, mask_l, mask_r).astype(jnp.bfloat16)
                y_s[n] = y2
                _accum_stats(y2, n, s2_s, ss2_s)

        @pl.when(jnp.logical_and(p == 2, t == 0))
        def _fold2():
            _fold(s2_s, ss2_s, g2_ref, b2_ref, sc2_s, sh2_s, cnt, eps)

        @pl.when(p == 2)
        def _phase3():
            for j in range(B):
                n = t * B + j
                o_ref[j] = jnp.maximum(
                    y_s[n].astype(jnp.float32) * sc2_s[...] + sh2_s[...]
                    + xb_s[n].astype(jnp.float32), 0.0)

    return body


def kernel(x_ncl, w1, g1, b1, w2, g2, b2):
    N, C, L = x_ncl.shape
    P = w1.shape[1]

    x_f = x_ncl.astype(jnp.float32)
    w1b = w1.astype(jnp.bfloat16)
    w2b = w2.astype(jnp.bfloat16)
    g1c = g1.astype(jnp.float32).reshape(P, 1)
    b1c = b1.astype(jnp.float32).reshape(P, 1)
    g2c = g2.astype(jnp.float32).reshape(P, 1)
    b2c = b2.astype(jnp.float32).reshape(P, 1)

    B = 8

    cparams = pltpu.CompilerParams(
        dimension_semantics=("arbitrary", "arbitrary"),
        vmem_limit_bytes=60 * 1024 * 1024)

    x_spec = pl.BlockSpec(
        (B, C, L), lambda p, t: (jnp.where(p == 0, t, 0), 0, 0))
    o_spec = pl.BlockSpec(
        (B, P, L), lambda p, t: (jnp.where(p == 2, t, 0), 0, 0))
    vec_spec = pl.BlockSpec((P, 1), lambda p, t: (0, 0))

    def w_spec(c_in):
        return pl.BlockSpec((3, P, c_in), lambda p, t: (0, 0, 0))

    stat_acc = pltpu.VMEM((P, 128), jnp.float32)
    stat = pltpu.VMEM((P, 1), jnp.float32)

    out = pl.pallas_call(
        _make_fused_kernel(N, L, B, _EPS),
        grid=(3, N // B),
        in_specs=[x_spec, vec_spec, vec_spec, vec_spec, vec_spec,
                  w_spec(C), w_spec(P)],
        out_specs=o_spec,
        out_shape=jax.ShapeDtypeStruct((N, P, L), jnp.float32),
        scratch_shapes=[
            pltpu.VMEM((N, C, L), jnp.bfloat16),
            pltpu.VMEM((N, P, L), jnp.bfloat16),
            stat_acc, stat_acc, stat_acc, stat_acc,
            stat, stat, stat, stat,
        ],
        compiler_params=cparams,
    )(x_f, g1c, b1c, g2c, b2c, w1b, w2b)

    return out


# speedup vs baseline: 1.0823x; 1.0165x over previous
import jax
import jax.numpy as jnp
from jax import lax
from jax.experimental import pallas as pl
from jax.experimental.pallas import tpu as pltpu

_EPS = 1e-5


def _conv3(h, w_ref):
    L = h.shape[1]
    pos = lax.broadcasted_iota(jnp.int32, (1, L), 1)
    zero = jnp.zeros((), h.dtype)
    h_m1 = jnp.where(pos != 0, pltpu.roll(h, 1, axis=1), zero)
    h_p1 = jnp.where(pos != (L - 1), pltpu.roll(h, L - 1, axis=1), zero)
    out = jnp.dot(w_ref[0], h_m1, preferred_element_type=jnp.float32)
    out += jnp.dot(w_ref[1], h, preferred_element_type=jnp.float32)
    out += jnp.dot(w_ref[2], h_p1, preferred_element_type=jnp.float32)
    return out


def _accum_stats(y, n, s_ref, ss_ref):
    sl = s_ref.shape[1]
    ssq = y * y
    s = y[:, :sl]
    ss = ssq[:, :sl]
    for k in range(1, y.shape[1] // sl):
        s = s + y[:, k * sl:(k + 1) * sl]
        ss = ss + ssq[:, k * sl:(k + 1) * sl]
    s32 = s.astype(jnp.float32)
    ss32 = ss.astype(jnp.float32)

    @pl.when(n == 0)
    def _():
        s_ref[...] = s32
        ss_ref[...] = ss32

    @pl.when(n != 0)
    def _():
        s_ref[...] += s32
        ss_ref[...] += ss32


def _fold(s_ref, ss_ref, g_ref, b_ref, sc_ref, sh_ref, count, eps):
    mean = jnp.sum(s_ref[...], axis=1, keepdims=True) / count
    ex2 = jnp.sum(ss_ref[...], axis=1, keepdims=True) / count
    var = ex2 - mean * mean
    scale = g_ref[...] * lax.rsqrt(var + eps)
    sc_ref[...] = scale
    sh_ref[...] = b_ref[...] - mean * scale


def _make_fused_kernel(N, L, B, eps):
    cnt = float(N * L)

    def body(x_ref, g1_ref, b1_ref, g2_ref, b2_ref, w1_ref, w2_ref,
             o_ref,
             xb_s, y_s, s_s, ss_s, s2_s, ss2_s,
             sc1_s, sh1_s, sc2_s, sh2_s):
        p = pl.program_id(0)
        t = pl.program_id(1)

        @pl.when(p == 0)
        def _phase1():
            for j in range(B):
                n = t * B + j
                xb = x_ref[j].astype(jnp.bfloat16)
                xb_s[n] = xb
                y1 = _conv3(xb, w1_ref).astype(jnp.bfloat16)
                y_s[n] = y1
                _accum_stats(y1, n, s_s, ss_s)

        @pl.when(jnp.logical_and(p == 1, t == 0))
        def _fold1():
            _fold(s_s, ss_s, g1_ref, b1_ref, sc1_s, sh1_s, cnt, eps)

        @pl.when(p == 1)
        def _phase2():
            sc = sc1_s[...].astype(jnp.bfloat16)
            sh = sh1_s[...].astype(jnp.bfloat16)
            zero = jnp.zeros((), jnp.bfloat16)
            for j in range(B):
                n = t * B + j
                h = jnp.maximum(y_s[n] * sc + sh, zero)
                y2 = _conv3(h, w2_ref).astype(jnp.bfloat16)
                y_s[n] = y2
                _accum_stats(y2, n, s2_s, ss2_s)

        @pl.when(jnp.logical_and(p == 2, t == 0))
        def _fold2():
            _fold(s2_s, ss2_s, g2_ref, b2_ref, sc2_s, sh2_s, cnt, eps)

        @pl.when(p == 2)
        def _phase3():
            for j in range(B):
                n = t * B + j
                o_ref[j] = jnp.maximum(
                    y_s[n].astype(jnp.float32) * sc2_s[...] + sh2_s[...]
                    + xb_s[n].astype(jnp.float32), 0.0)

    return body


def kernel(x_ncl, w1, g1, b1, w2, g2, b2):
    N, C, L = x_ncl.shape
    P = w1.shape[1]

    x_f = x_ncl.astype(jnp.float32)
    w1b = w1.astype(jnp.bfloat16)
    w2b = w2.astype(jnp.bfloat16)
    g1c = g1.astype(jnp.float32).reshape(P, 1)
    b1c = b1.astype(jnp.float32).reshape(P, 1)
    g2c = g2.astype(jnp.float32).reshape(P, 1)
    b2c = b2.astype(jnp.float32).reshape(P, 1)

    B = 8

    cparams = pltpu.CompilerParams(
        dimension_semantics=("arbitrary", "arbitrary"),
        vmem_limit_bytes=60 * 1024 * 1024)

    x_spec = pl.BlockSpec(
        (B, C, L), lambda p, t: (jnp.where(p == 0, t, 0), 0, 0))
    o_spec = pl.BlockSpec(
        (B, P, L), lambda p, t: (jnp.where(p == 2, t, 0), 0, 0))
    vec_spec = pl.BlockSpec((P, 1), lambda p, t: (0, 0))

    def w_spec(c_in):
        return pl.BlockSpec((3, P, c_in), lambda p, t: (0, 0, 0))

    stat_acc = pltpu.VMEM((P, 128), jnp.float32)
    stat = pltpu.VMEM((P, 1), jnp.float32)

    out = pl.pallas_call(
        _make_fused_kernel(N, L, B, _EPS),
        grid=(3, N // B),
        in_specs=[x_spec, vec_spec, vec_spec, vec_spec, vec_spec,
                  w_spec(C), w_spec(P)],
        out_specs=o_spec,
        out_shape=jax.ShapeDtypeStruct((N, P, L), jnp.float32),
        scratch_shapes=[
            pltpu.VMEM((N, C, L), jnp.bfloat16),
            pltpu.VMEM((N, P, L), jnp.bfloat16),
            stat_acc, stat_acc, stat_acc, stat_acc,
            stat, stat, stat, stat,
        ],
        compiler_params=cparams,
    )(x_f, g1c, b1c, g2c, b2c, w1b, w2b)

    return out
